# baseline (speedup 1.0000x reference)
import functools

import jax
import jax.numpy as jnp
from jax import lax
from jax.experimental import pallas as pl
from jax.experimental.pallas import tpu as pltpu
from jax.experimental.pallas import tpu_sc as plsc

N = 10000
F = 128
FA = 16
E = 160000
NC, NS = 2, 16
NW = NC * NS
CH = 64
EPT = 5120
EPAD = EPT * NW
NCHUNK = EPT // CH
NA = 10240
WCH = 64
NZCH = NA // (WCH * NS)
NBUF = 2
EROWS = EPAD // CH
C0, C1 = 46, 114
SLAB = max(C0, C1)


def _sc_pass(tx, ta, srcp, dstp, z128, z16):
    mesh = plsc.VectorSubcoreMesh(core_axis_name="c", subcore_axis_name="s")

    @functools.partial(
        pl.kernel,
        out_type=(
            jax.ShapeDtypeStruct((NC, NA, F), jnp.float32),
            jax.ShapeDtypeStruct((NC, NA, FA), jnp.float32),
        ),
        mesh=mesh,
        scratch_types=[
            pltpu.VMEM((SLAB, CH), jnp.int32),
            pltpu.VMEM((SLAB, CH), jnp.int32),
            [pltpu.VMEM((CH, F), jnp.float32)] * NBUF,
            [pltpu.VMEM((CH, FA), jnp.float32)] * NBUF,
            pltpu.VMEM_SHARED((NA, F), jnp.float32),
            pltpu.VMEM_SHARED((NA, FA), jnp.float32),
            [pltpu.SemaphoreType.DMA] * NBUF,
            [pltpu.SemaphoreType.DMA] * NBUF,
        ],
        compiler_params=pltpu.CompilerParams(use_tc_tiling_on_sc=False),
    )
    def k(tx_h, ta_h, srcp_h, dstp_h, z128_h, z16_h, ox_h, oa_h,
          sidx, didx, gbx, gba, accx, acca, gsem, ssem):
        cid = lax.axis_index("c")
        sid = lax.axis_index("s")
        wid = sid * NC + cid
        pltpu.sync_copy(z128_h, gbx[0])
        pltpu.sync_copy(z16_h, gba[0])
        for j in range(NZCH):
            rj = (sid * NZCH + j) * WCH
            pltpu.sync_copy(gbx[0].at[pl.ds(0, WCH), :],
                            accx.at[pl.ds(rj, WCH), :])
            pltpu.sync_copy(gba[0].at[pl.ds(0, WCH), :],
                            acca.at[pl.ds(rj, WCH), :])
        plsc.subcore_barrier()
        row0 = jnp.where(cid == 0, sid * C0, NS * C0 + sid * C1)
        nch = jnp.where(cid == 0, C0, C1)
        nit = jnp.where(cid == 0, C0 // NBUF, C1 // NBUF)
        pltpu.sync_copy(srcp_h.at[pl.ds(row0, SLAB), :], sidx)
        pltpu.sync_copy(dstp_h.at[pl.ds(row0, SLAB), :], didx)

        def g_desc(i, b):
            ic = jnp.minimum(i, nch - 1)
            return (pltpu.make_async_copy(tx_h.at[sidx.at[ic]], gbx[b], gsem[b]),
                    pltpu.make_async_copy(ta_h.at[sidx.at[ic]], gba[b], gsem[b]))

        def s_desc(i, b):
            return (pltpu.make_async_copy(gbx[b], accx.at[didx.at[i]], ssem[b]),
                    pltpu.make_async_copy(gba[b], acca.at[didx.at[i]], ssem[b]))

        for b in range(NBUF):
            for d in g_desc(b, b):
                d.start()

        def itr(j, carry):
            i0 = j * NBUF
            for b in range(NBUF):
                for d in g_desc(i0 + b, b):
                    d.wait()
                for d in s_desc(i0 + b, b):
                    d.start(add=True)
            for b in range(NBUF):
                for d in s_desc(i0 + b, b):
                    d.wait()
                for d in g_desc(i0 + NBUF + b, b):
                    d.start()
            return carry

        lax.fori_loop(0, nit, itr, 0)
        for b in range(NBUF):
            for d in g_desc(0, b):
                d.wait()
        plsc.subcore_barrier()
        for j in range(NZCH):
            rj = (sid * NZCH + j) * WCH
            pltpu.sync_copy(accx.at[pl.ds(rj, WCH), :],
                            gbx[0].at[pl.ds(0, WCH), :])
            pltpu.sync_copy(gbx[0].at[pl.ds(0, WCH), :],
                            ox_h.at[cid, pl.ds(rj, WCH), :])
            pltpu.sync_copy(acca.at[pl.ds(rj, WCH), :],
                            gba[0].at[pl.ds(0, WCH), :])
            pltpu.sync_copy(gba[0].at[pl.ds(0, WCH), :],
                            oa_h.at[cid, pl.ds(rj, WCH), :])

    return k(tx, ta, srcp, dstp, z128, z16)


def _combine(px, pa):
    R = 1024

    def body(px_ref, pa_ref, ox_ref, oa_ref):
        ox_ref[...] = px_ref[0] + px_ref[1]
        oa_ref[...] = pa_ref[0] + pa_ref[1]

    return pl.pallas_call(
        body,
        grid=(NA // R,),
        in_specs=[
            pl.BlockSpec((NC, R, F), lambda i: (0, i, 0)),
            pl.BlockSpec((NC, R, FA), lambda i: (0, i, 0)),
        ],
        out_specs=[
            pl.BlockSpec((R, F), lambda i: (i, 0)),
            pl.BlockSpec((R, FA), lambda i: (i, 0)),
        ],
        out_shape=[
            jax.ShapeDtypeStruct((NA, F), jnp.float32),
            jax.ShapeDtypeStruct((NA, FA), jnp.float32),
        ],
    )(px, pa)


def _final(xs, s1x, s2x, scal,
           Wf0_0, bf0_0, Wf1_0, bf1_0, Wc_0, bc_0,
           Wf0_1, bf0_1, Wf1_1, bf1_1, Wc_1, bc_1):
    def body(xs_ref, s1_ref, s2_ref, scal_ref,
             wf00, bf00, wf10, bf10, wc0, bc0,
             wf01, bf01, wf11, bf11, wc1, bc1, o_ref):
        mm = lambda a, b: jnp.dot(a, b, preferred_element_type=jnp.float32)
        Wf00 = wf00[...]; Wf10 = wf10[...]; Wc0 = wc0[...]
        Wf01 = wf01[...]; Wf11 = wf11[...]; Wc1 = wc1[...]
        m1 = mm(Wc0, Wf01)
        Wch = mm(Wf00, m1)
        bv1 = mm(bf00[...], m1)
        bv2 = mm(bc0[...], Wf01) + bf01[...]

        xs_v = xs_ref[...]
        h0s = mm(xs_v, Wf00) + bf00[...]
        x1 = mm(xs_v, Wf10) + bf10[...]
        u = mm(x1 - h0s, Wc0)
        v = mm(u, Wf01)

        A1 = s1_ref[...]
        A2 = s2_ref[...]

        t_rows = []
        v_rows = []
        for b in range(4):
            deg_b = scal_ref[32 + b]
            base0 = mm(A1[b:b + 1, :], Wf00) + deg_b * bf00[...]
            H1 = mm(base0, Wc0) + bc0[...]
            Xb = mm(H1, Wf01) + bf01[...]
            cs = scal_ref[b * 4 + b]
            ub = u[b:b + 1, :]
            vb = v[b:b + 1, :]
            wb = mm(H1 + cs * ub, Wf11) + bf11[...]
            t_rows.append(wb - Xb - cs * vb)
            v_rows.append(vb)

        base1 = []
        for i in range(4):
            base1.append(mm(A2[i:i + 1, :], Wch)
                         + scal_ref[36 + i] * bv1
                         + scal_ref[32 + i] * bv2)

        row = 0
        for m in range(2):
            i0, i1 = 2 * m, 2 * m + 1
            for (b, i) in ((i0, i0), (i1, i1), (i0, i1), (i1, i0)):
                r = (base1[i]
                     + scal_ref[16 + i * 4 + b] * v_rows[b]
                     + scal_ref[i * 4 + b] * t_rows[b])
                o_ref[row:row + 1, :] = mm(r, Wc1) + bc1[...]
                row += 1

    vspec = pl.BlockSpec(memory_space=pltpu.VMEM)
    sspec = pl.BlockSpec(memory_space=pltpu.SMEM)
    return pl.pallas_call(
        body,
        in_specs=[vspec, vspec, vspec, sspec] + [vspec] * 12,
        out_specs=pl.BlockSpec(memory_space=pltpu.VMEM),
        out_shape=jax.ShapeDtypeStruct((8, F), jnp.float32),
    )(xs, s1x, s2x, scal,
      Wf0_0, bf0_0, Wf1_0, bf1_0, Wc_0, bc_0,
      Wf0_1, bf0_1, Wf1_1, bf1_1, Wc_1, bc_1)


def kernel(x, edge_index, pos,
           Wf0_0, bf0_0, Wf1_0, bf1_0, Wc_0, bc_0,
           Wf0_1, bf0_1, Wf1_1, bf1_1, Wc_1, bc_1):
    sel = pos.reshape(4).astype(jnp.int32)
    src = edge_index[0]
    dst = edge_index[1]
    npad = EROWS * CH - E
    srcp = jnp.concatenate([src, jnp.zeros((npad,), jnp.int32)]).reshape(EROWS, CH)
    dstp = jnp.concatenate([dst, jnp.full((npad,), N, jnp.int32)]).reshape(EROWS, CH)
    ta = jnp.zeros((N, FA), jnp.float32)
    ta = ta.at[:, 4].set(1.0)
    ta = ta.at[sel, jnp.arange(4)].set(1.0)
    z128 = jnp.zeros((CH, F), jnp.float32)
    z16 = jnp.zeros((CH, FA), jnp.float32)

    p1x, p1a = _sc_pass(x, ta, srcp, dstp, z128, z16)
    s1x_full, s1a_full = _combine(p1x, p1a)
    p2x, p2a = _sc_pass(s1x_full, s1a_full, srcp, dstp, z128, z16)

    xs = x[sel]
    s1x = s1x_full[sel]
    s1a = s1a_full[sel]
    s2x = p2x[0][sel] + p2x[1][sel]
    s2a = p2a[0][sel] + p2a[1][sel]
    scal = jnp.concatenate([
        s1a[:, :4].reshape(16),
        s2a[:, :4].reshape(16),
        s1a[:, 4],
        s2a[:, 4],
        jnp.zeros((8,), jnp.float32),
    ])

    b = lambda a: a.reshape(1, F)
    out8 = _final(xs, s1x, s2x, scal,
                  Wf0_0, b(bf0_0), Wf1_0, b(bf1_0), Wc_0, b(bc_0),
                  Wf0_1, b(bf0_1), Wf1_1, b(bf1_1), Wc_1, b(bc_1))
    return out8.reshape(2, 2, 2, F)


# speedup vs baseline: 1.1836x; 1.1836x over previous
import functools

import jax
import jax.numpy as jnp
from jax import lax
from jax.experimental import pallas as pl
from jax.experimental.pallas import tpu as pltpu
from jax.experimental.pallas import tpu_sc as plsc

N = 10000
F = 128
FA = 16
E = 160000
NC, NS = 2, 16
NW = NC * NS
CH = 64
EPT = 5120
EPAD = EPT * NW
NCHUNK = EPT // CH
NA = 10240
WCH = 64
NZCH = NA // (WCH * NS)
NBUF = 2
EROWS = EPAD // CH
C0, C1 = 114, 46
SLAB = max(C0, C1)


def _sc_pass(tx, ta, srcp, dstp, z128, z16):
    mesh = plsc.VectorSubcoreMesh(core_axis_name="c", subcore_axis_name="s")

    @functools.partial(
        pl.kernel,
        out_type=(
            jax.ShapeDtypeStruct((NC, NA, F), jnp.float32),
            jax.ShapeDtypeStruct((NC, NA, FA), jnp.float32),
        ),
        mesh=mesh,
        scratch_types=[
            pltpu.VMEM((SLAB, CH), jnp.int32),
            pltpu.VMEM((SLAB, CH), jnp.int32),
            [pltpu.VMEM((CH, F), jnp.float32)] * NBUF,
            [pltpu.VMEM((CH, FA), jnp.float32)] * NBUF,
            pltpu.VMEM_SHARED((NA, F), jnp.float32),
            pltpu.VMEM_SHARED((NA, FA), jnp.float32),
            [pltpu.SemaphoreType.DMA] * NBUF,
            [pltpu.SemaphoreType.DMA] * NBUF,
        ],
        compiler_params=pltpu.CompilerParams(use_tc_tiling_on_sc=False),
    )
    def k(tx_h, ta_h, srcp_h, dstp_h, z128_h, z16_h, ox_h, oa_h,
          sidx, didx, gbx, gba, accx, acca, gsem, ssem):
        cid = lax.axis_index("c")
        sid = lax.axis_index("s")
        wid = sid * NC + cid
        pltpu.sync_copy(z128_h, gbx[0])
        pltpu.sync_copy(z16_h, gba[0])
        for j in range(NZCH):
            rj = (sid * NZCH + j) * WCH
            pltpu.sync_copy(gbx[0].at[pl.ds(0, WCH), :],
                            accx.at[pl.ds(rj, WCH), :])
            pltpu.sync_copy(gba[0].at[pl.ds(0, WCH), :],
                            acca.at[pl.ds(rj, WCH), :])
        plsc.subcore_barrier()
        row0 = jnp.where(cid == 0, sid * C0, NS * C0 + sid * C1)
        nch = jnp.where(cid == 0, C0, C1)
        nit = jnp.where(cid == 0, C0 // NBUF, C1 // NBUF)
        pltpu.sync_copy(srcp_h.at[pl.ds(row0, SLAB), :], sidx)
        pltpu.sync_copy(dstp_h.at[pl.ds(row0, SLAB), :], didx)

        def g_desc(i, b):
            ic = jnp.minimum(i, nch - 1)
            return (pltpu.make_async_copy(tx_h.at[sidx.at[ic]], gbx[b], gsem[b]),
                    pltpu.make_async_copy(ta_h.at[sidx.at[ic]], gba[b], gsem[b]))

        def s_desc(i, b):
            return (pltpu.make_async_copy(gbx[b], accx.at[didx.at[i]], ssem[b]),
                    pltpu.make_async_copy(gba[b], acca.at[didx.at[i]], ssem[b]))

        for b in range(NBUF):
            for d in g_desc(b, b):
                d.start()

        def itr(j, carry):
            i0 = j * NBUF
            for b in range(NBUF):
                for d in g_desc(i0 + b, b):
                    d.wait()
                for d in s_desc(i0 + b, b):
                    d.start(add=True)
            for b in range(NBUF):
                for d in s_desc(i0 + b, b):
                    d.wait()
                for d in g_desc(i0 + NBUF + b, b):
                    d.start()
            return carry

        lax.fori_loop(0, nit, itr, 0)
        for b in range(NBUF):
            for d in g_desc(0, b):
                d.wait()
        plsc.subcore_barrier()
        for j in range(NZCH):
            rj = (sid * NZCH + j) * WCH
            pltpu.sync_copy(accx.at[pl.ds(rj, WCH), :],
                            gbx[0].at[pl.ds(0, WCH), :])
            pltpu.sync_copy(gbx[0].at[pl.ds(0, WCH), :],
                            ox_h.at[cid, pl.ds(rj, WCH), :])
            pltpu.sync_copy(acca.at[pl.ds(rj, WCH), :],
                            gba[0].at[pl.ds(0, WCH), :])
            pltpu.sync_copy(gba[0].at[pl.ds(0, WCH), :],
                            oa_h.at[cid, pl.ds(rj, WCH), :])

    return k(tx, ta, srcp, dstp, z128, z16)


def _combine(px, pa):
    R = 1024

    def body(px_ref, pa_ref, ox_ref, oa_ref):
        ox_ref[...] = px_ref[0] + px_ref[1]
        oa_ref[...] = pa_ref[0] + pa_ref[1]

    return pl.pallas_call(
        body,
        grid=(NA // R,),
        in_specs=[
            pl.BlockSpec((NC, R, F), lambda i: (0, i, 0)),
            pl.BlockSpec((NC, R, FA), lambda i: (0, i, 0)),
        ],
        out_specs=[
            pl.BlockSpec((R, F), lambda i: (i, 0)),
            pl.BlockSpec((R, FA), lambda i: (i, 0)),
        ],
        out_shape=[
            jax.ShapeDtypeStruct((NA, F), jnp.float32),
            jax.ShapeDtypeStruct((NA, FA), jnp.float32),
        ],
    )(px, pa)


def _final(xs, s1x, s2x, scal,
           Wf0_0, bf0_0, Wf1_0, bf1_0, Wc_0, bc_0,
           Wf0_1, bf0_1, Wf1_1, bf1_1, Wc_1, bc_1):
    def body(xs_ref, s1_ref, s2_ref, scal_ref,
             wf00, bf00, wf10, bf10, wc0, bc0,
             wf01, bf01, wf11, bf11, wc1, bc1, o_ref):
        mm = lambda a, b: jnp.dot(a, b, preferred_element_type=jnp.float32)
        Wf00 = wf00[...]; Wf10 = wf10[...]; Wc0 = wc0[...]
        Wf01 = wf01[...]; Wf11 = wf11[...]; Wc1 = wc1[...]
        m1 = mm(Wc0, Wf01)
        Wch = mm(Wf00, m1)
        bv1 = mm(bf00[...], m1)
        bv2 = mm(bc0[...], Wf01) + bf01[...]

        xs_v = xs_ref[...]
        h0s = mm(xs_v, Wf00) + bf00[...]
        x1 = mm(xs_v, Wf10) + bf10[...]
        u = mm(x1 - h0s, Wc0)
        v = mm(u, Wf01)

        A1 = s1_ref[...]
        A2 = s2_ref[...]

        t_rows = []
        v_rows = []
        for b in range(4):
            deg_b = scal_ref[32 + b]
            base0 = mm(A1[b:b + 1, :], Wf00) + deg_b * bf00[...]
            H1 = mm(base0, Wc0) + bc0[...]
            Xb = mm(H1, Wf01) + bf01[...]
            cs = scal_ref[b * 4 + b]
            ub = u[b:b + 1, :]
            vb = v[b:b + 1, :]
            wb = mm(H1 + cs * ub, Wf11) + bf11[...]
            t_rows.append(wb - Xb - cs * vb)
            v_rows.append(vb)

        base1 = []
        for i in range(4):
            base1.append(mm(A2[i:i + 1, :], Wch)
                         + scal_ref[36 + i] * bv1
                         + scal_ref[32 + i] * bv2)

        row = 0
        for m in range(2):
            i0, i1 = 2 * m, 2 * m + 1
            for (b, i) in ((i0, i0), (i1, i1), (i0, i1), (i1, i0)):
                r = (base1[i]
                     + scal_ref[16 + i * 4 + b] * v_rows[b]
                     + scal_ref[i * 4 + b] * t_rows[b])
                o_ref[row:row + 1, :] = mm(r, Wc1) + bc1[...]
                row += 1

    vspec = pl.BlockSpec(memory_space=pltpu.VMEM)
    sspec = pl.BlockSpec(memory_space=pltpu.SMEM)
    return pl.pallas_call(
        body,
        in_specs=[vspec, vspec, vspec, sspec] + [vspec] * 12,
        out_specs=pl.BlockSpec(memory_space=pltpu.VMEM),
        out_shape=jax.ShapeDtypeStruct((8, F), jnp.float32),
    )(xs, s1x, s2x, scal,
      Wf0_0, bf0_0, Wf1_0, bf1_0, Wc_0, bc_0,
      Wf0_1, bf0_1, Wf1_1, bf1_1, Wc_1, bc_1)


def kernel(x, edge_index, pos,
           Wf0_0, bf0_0, Wf1_0, bf1_0, Wc_0, bc_0,
           Wf0_1, bf0_1, Wf1_1, bf1_1, Wc_1, bc_1):
    sel = pos.reshape(4).astype(jnp.int32)
    src = edge_index[0]
    dst = edge_index[1]
    npad = EROWS * CH - E
    srcp = jnp.concatenate([src, jnp.zeros((npad,), jnp.int32)]).reshape(EROWS, CH)
    dstp = jnp.concatenate([dst, jnp.full((npad,), N, jnp.int32)]).reshape(EROWS, CH)
    ta = jnp.zeros((N, FA), jnp.float32)
    ta = ta.at[:, 4].set(1.0)
    ta = ta.at[sel, jnp.arange(4)].set(1.0)
    z128 = jnp.zeros((CH, F), jnp.float32)
    z16 = jnp.zeros((CH, FA), jnp.float32)

    p1x, p1a = _sc_pass(x, ta, srcp, dstp, z128, z16)
    s1x_full, s1a_full = _combine(p1x, p1a)
    p2x, p2a = _sc_pass(s1x_full, s1a_full, srcp, dstp, z128, z16)

    xs = x[sel]
    s1x = s1x_full[sel]
    s1a = s1a_full[sel]
    s2x = p2x[0][sel] + p2x[1][sel]
    s2a = p2a[0][sel] + p2a[1][sel]
    scal = jnp.concatenate([
        s1a[:, :4].reshape(16),
        s2a[:, :4].reshape(16),
        s1a[:, 4],
        s2a[:, 4],
        jnp.zeros((8,), jnp.float32),
    ])

    b = lambda a: a.reshape(1, F)
    out8 = _final(xs, s1x, s2x, scal,
                  Wf0_0, b(bf0_0), Wf1_0, b(bf1_0), Wc_0, b(bc_0),
                  Wf0_1, b(bf0_1), Wf1_1, b(bf1_1), Wc_1, b(bc_1))
    return out8.reshape(2, 2, 2, F)


# speedup vs baseline: 1.2136x; 1.0253x over previous
import functools

import jax
import jax.numpy as jnp
from jax import lax
from jax.experimental import pallas as pl
from jax.experimental.pallas import tpu as pltpu
from jax.experimental.pallas import tpu_sc as plsc

N = 10000
F = 128
FA = 16
E = 160000
NC, NS = 2, 16
NW = NC * NS
CH = 64
EPT = 5120
EPAD = EPT * NW
NCHUNK = EPT // CH
NA = 10240
WCH = 64
NZCH = NA // (WCH * NS)
NBUF = 2
EROWS = EPAD // CH
C0, C1 = 132, 28
SLAB = max(C0, C1)


def _sc_pass(tx, ta, srcp, dstp, z128, z16):
    mesh = plsc.VectorSubcoreMesh(core_axis_name="c", subcore_axis_name="s")

    @functools.partial(
        pl.kernel,
        out_type=(
            jax.ShapeDtypeStruct((NC, NA, F), jnp.float32),
            jax.ShapeDtypeStruct((NC, NA, FA), jnp.float32),
        ),
        mesh=mesh,
        scratch_types=[
            pltpu.VMEM((SLAB, CH), jnp.int32),
            pltpu.VMEM((SLAB, CH), jnp.int32),
            [pltpu.VMEM((CH, F), jnp.float32)] * NBUF,
            [pltpu.VMEM((CH, FA), jnp.float32)] * NBUF,
            pltpu.VMEM_SHARED((NA, F), jnp.float32),
            pltpu.VMEM_SHARED((NA, FA), jnp.float32),
            [pltpu.SemaphoreType.DMA] * NBUF,
            [pltpu.SemaphoreType.DMA] * NBUF,
        ],
        compiler_params=pltpu.CompilerParams(use_tc_tiling_on_sc=False),
    )
    def k(tx_h, ta_h, srcp_h, dstp_h, z128_h, z16_h, ox_h, oa_h,
          sidx, didx, gbx, gba, accx, acca, gsem, ssem):
        cid = lax.axis_index("c")
        sid = lax.axis_index("s")
        wid = sid * NC + cid
        pltpu.sync_copy(z128_h, gbx[0])
        pltpu.sync_copy(z16_h, gba[0])
        for j in range(NZCH):
            rj = (sid * NZCH + j) * WCH
            pltpu.sync_copy(gbx[0].at[pl.ds(0, WCH), :],
                            accx.at[pl.ds(rj, WCH), :])
            pltpu.sync_copy(gba[0].at[pl.ds(0, WCH), :],
                            acca.at[pl.ds(rj, WCH), :])
        plsc.subcore_barrier()
        row0 = jnp.where(cid == 0, sid * C0, NS * C0 + sid * C1)
        nch = jnp.where(cid == 0, C0, C1)
        nit = jnp.where(cid == 0, C0 // NBUF, C1 // NBUF)
        pltpu.sync_copy(srcp_h.at[pl.ds(row0, SLAB), :], sidx)
        pltpu.sync_copy(dstp_h.at[pl.ds(row0, SLAB), :], didx)

        def g_desc(i, b):
            ic = jnp.minimum(i, nch - 1)
            return (pltpu.make_async_copy(tx_h.at[sidx.at[ic]], gbx[b], gsem[b]),
                    pltpu.make_async_copy(ta_h.at[sidx.at[ic]], gba[b], gsem[b]))

        def s_desc(i, b):
            return (pltpu.make_async_copy(gbx[b], accx.at[didx.at[i]], ssem[b]),
                    pltpu.make_async_copy(gba[b], acca.at[didx.at[i]], ssem[b]))

        for b in range(NBUF):
            for d in g_desc(b, b):
                d.start()

        def itr(j, carry):
            i0 = j * NBUF
            for b in range(NBUF):
                for d in g_desc(i0 + b, b):
                    d.wait()
                for d in s_desc(i0 + b, b):
                    d.start(add=True)
            for b in range(NBUF):
                for d in s_desc(i0 + b, b):
                    d.wait()
                for d in g_desc(i0 + NBUF + b, b):
                    d.start()
            return carry

        lax.fori_loop(0, nit, itr, 0)
        for b in range(NBUF):
            for d in g_desc(0, b):
                d.wait()
        plsc.subcore_barrier()
        for j in range(NZCH):
            rj = (sid * NZCH + j) * WCH
            pltpu.sync_copy(accx.at[pl.ds(rj, WCH), :],
                            gbx[0].at[pl.ds(0, WCH), :])
            pltpu.sync_copy(gbx[0].at[pl.ds(0, WCH), :],
                            ox_h.at[cid, pl.ds(rj, WCH), :])
            pltpu.sync_copy(acca.at[pl.ds(rj, WCH), :],
                            gba[0].at[pl.ds(0, WCH), :])
            pltpu.sync_copy(gba[0].at[pl.ds(0, WCH), :],
                            oa_h.at[cid, pl.ds(rj, WCH), :])

    return k(tx, ta, srcp, dstp, z128, z16)


def _combine(px, pa):
    R = 1024

    def body(px_ref, pa_ref, ox_ref, oa_ref):
        ox_ref[...] = px_ref[0] + px_ref[1]
        oa_ref[...] = pa_ref[0] + pa_ref[1]

    return pl.pallas_call(
        body,
        grid=(NA // R,),
        in_specs=[
            pl.BlockSpec((NC, R, F), lambda i: (0, i, 0)),
            pl.BlockSpec((NC, R, FA), lambda i: (0, i, 0)),
        ],
        out_specs=[
            pl.BlockSpec((R, F), lambda i: (i, 0)),
            pl.BlockSpec((R, FA), lambda i: (i, 0)),
        ],
        out_shape=[
            jax.ShapeDtypeStruct((NA, F), jnp.float32),
            jax.ShapeDtypeStruct((NA, FA), jnp.float32),
        ],
    )(px, pa)


def _final(xs, s1x, s2x, scal,
           Wf0_0, bf0_0, Wf1_0, bf1_0, Wc_0, bc_0,
           Wf0_1, bf0_1, Wf1_1, bf1_1, Wc_1, bc_1):
    def body(xs_ref, s1_ref, s2_ref, scal_ref,
             wf00, bf00, wf10, bf10, wc0, bc0,
             wf01, bf01, wf11, bf11, wc1, bc1, o_ref):
        mm = lambda a, b: jnp.dot(a, b, preferred_element_type=jnp.float32)
        Wf00 = wf00[...]; Wf10 = wf10[...]; Wc0 = wc0[...]
        Wf01 = wf01[...]; Wf11 = wf11[...]; Wc1 = wc1[...]
        m1 = mm(Wc0, Wf01)
        Wch = mm(Wf00, m1)
        bv1 = mm(bf00[...], m1)
        bv2 = mm(bc0[...], Wf01) + bf01[...]

        xs_v = xs_ref[...]
        h0s = mm(xs_v, Wf00) + bf00[...]
        x1 = mm(xs_v, Wf10) + bf10[...]
        u = mm(x1 - h0s, Wc0)
        v = mm(u, Wf01)

        A1 = s1_ref[...]
        A2 = s2_ref[...]

        t_rows = []
        v_rows = []
        for b in range(4):
            deg_b = scal_ref[32 + b]
            base0 = mm(A1[b:b + 1, :], Wf00) + deg_b * bf00[...]
            H1 = mm(base0, Wc0) + bc0[...]
            Xb = mm(H1, Wf01) + bf01[...]
            cs = scal_ref[b * 4 + b]
            ub = u[b:b + 1, :]
            vb = v[b:b + 1, :]
            wb = mm(H1 + cs * ub, Wf11) + bf11[...]
            t_rows.append(wb - Xb - cs * vb)
            v_rows.append(vb)

        base1 = []
        for i in range(4):
            base1.append(mm(A2[i:i + 1, :], Wch)
                         + scal_ref[36 + i] * bv1
                         + scal_ref[32 + i] * bv2)

        row = 0
        for m in range(2):
            i0, i1 = 2 * m, 2 * m + 1
            for (b, i) in ((i0, i0), (i1, i1), (i0, i1), (i1, i0)):
                r = (base1[i]
                     + scal_ref[16 + i * 4 + b] * v_rows[b]
                     + scal_ref[i * 4 + b] * t_rows[b])
                o_ref[row:row + 1, :] = mm(r, Wc1) + bc1[...]
                row += 1

    vspec = pl.BlockSpec(memory_space=pltpu.VMEM)
    sspec = pl.BlockSpec(memory_space=pltpu.SMEM)
    return pl.pallas_call(
        body,
        in_specs=[vspec, vspec, vspec, sspec] + [vspec] * 12,
        out_specs=pl.BlockSpec(memory_space=pltpu.VMEM),
        out_shape=jax.ShapeDtypeStruct((8, F), jnp.float32),
    )(xs, s1x, s2x, scal,
      Wf0_0, bf0_0, Wf1_0, bf1_0, Wc_0, bc_0,
      Wf0_1, bf0_1, Wf1_1, bf1_1, Wc_1, bc_1)


def kernel(x, edge_index, pos,
           Wf0_0, bf0_0, Wf1_0, bf1_0, Wc_0, bc_0,
           Wf0_1, bf0_1, Wf1_1, bf1_1, Wc_1, bc_1):
    sel = pos.reshape(4).astype(jnp.int32)
    src = edge_index[0]
    dst = edge_index[1]
    npad = EROWS * CH - E
    srcp = jnp.concatenate([src, jnp.zeros((npad,), jnp.int32)]).reshape(EROWS, CH)
    dstp = jnp.concatenate([dst, jnp.full((npad,), N, jnp.int32)]).reshape(EROWS, CH)
    ta = jnp.zeros((N, FA), jnp.float32)
    ta = ta.at[:, 4].set(1.0)
    ta = ta.at[sel, jnp.arange(4)].set(1.0)
    z128 = jnp.zeros((CH, F), jnp.float32)
    z16 = jnp.zeros((CH, FA), jnp.float32)

    p1x, p1a = _sc_pass(x, ta, srcp, dstp, z128, z16)
    s1x_full, s1a_full = _combine(p1x, p1a)
    p2x, p2a = _sc_pass(s1x_full, s1a_full, srcp, dstp, z128, z16)

    xs = x[sel]
    s1x = s1x_full[sel]
    s1a = s1a_full[sel]
    s2x = p2x[0][sel] + p2x[1][sel]
    s2a = p2a[0][sel] + p2a[1][sel]
    scal = jnp.concatenate([
        s1a[:, :4].reshape(16),
        s2a[:, :4].reshape(16),
        s1a[:, 4],
        s2a[:, 4],
        jnp.zeros((8,), jnp.float32),
    ])

    b = lambda a: a.reshape(1, F)
    out8 = _final(xs, s1x, s2x, scal,
                  Wf0_0, b(bf0_0), Wf1_0, b(bf1_0), Wc_0, b(bc_0),
                  Wf0_1, b(bf0_1), Wf1_1, b(bf1_1), Wc_1, b(bc_1))
    return out8.reshape(2, 2, 2, F)
